# baseline (speedup 1.0000x reference)
import functools

import jax
import jax.numpy as jnp
from jax import lax
from jax.experimental import pallas as pl
from jax.experimental.pallas import tpu as pltpu

VOCAB = 100000
EMBED_DIM = 64
CONTEXT = 4
BATCH = 1024
HIDDEN = 128

VT = 2048
NT = (VOCAB + VT - 1) // VT


def _fused_body(embeds_ref, w1_ref, b1_ref, w2_ref, b2_ref, out_ref,
                h_ref, m_ref, s_ref):
    p = pl.program_id(0)
    t = pl.program_id(1)

    @pl.when((p == 0) & (t == 0))
    def _init():
        e = embeds_ref[...].astype(jnp.bfloat16)
        w1 = w1_ref[...].astype(jnp.bfloat16)
        h = lax.dot_general(e, w1, (((1,), (1,)), ((), ())),
                            preferred_element_type=jnp.float32)
        h_ref[...] = jnp.maximum(h + b1_ref[...], 0.0)
        m_ref[...] = jnp.full_like(m_ref, -jnp.inf)
        s_ref[...] = jnp.zeros_like(s_ref)

    h = h_ref[...].astype(jnp.bfloat16)
    w2 = w2_ref[...].astype(jnp.bfloat16)
    logits = lax.dot_general(h, w2, (((1,), (1,)), ((), ())),
                             preferred_element_type=jnp.float32)
    logits = logits + b2_ref[...]

    @pl.when(p == 0)
    def _stats():
        col = t * VT + lax.broadcasted_iota(jnp.int32, logits.shape, 1)
        lg = jnp.where(col < VOCAB, logits, -jnp.inf)
        tile_max = jnp.max(lg, axis=1, keepdims=True)
        m_old = m_ref[...]
        m_new = jnp.maximum(m_old, tile_max)
        s_ref[...] = (s_ref[...] * jnp.exp(m_old - m_new)
                      + jnp.sum(jnp.exp(lg - m_new), axis=1, keepdims=True))
        m_ref[...] = m_new

    @pl.when(p == 1)
    def _write():
        out_ref[...] = logits - m_ref[...] - jnp.log(s_ref[...])


def _fused_logsoftmax(embeds, W1, b1, b2_row, W2, *, interpret=False):
    return pl.pallas_call(
        _fused_body,
        grid=(2, NT),
        in_specs=[
            pl.BlockSpec((BATCH, EMBED_DIM * CONTEXT), lambda p, t: (0, 0)),
            pl.BlockSpec((HIDDEN, EMBED_DIM * CONTEXT), lambda p, t: (0, 0)),
            pl.BlockSpec((1, HIDDEN), lambda p, t: (0, 0)),
            pl.BlockSpec((VT, HIDDEN), lambda p, t: (t, 0)),
            pl.BlockSpec((1, VT), lambda p, t: (0, t)),
        ],
        out_specs=pl.BlockSpec((BATCH, VT), lambda p, t: (0, p * t)),
        out_shape=jax.ShapeDtypeStruct((BATCH, VOCAB), jnp.float32),
        scratch_shapes=[
            pltpu.VMEM((BATCH, HIDDEN), jnp.float32),
            pltpu.VMEM((BATCH, 1), jnp.float32),
            pltpu.VMEM((BATCH, 1), jnp.float32),
        ],
        compiler_params=pltpu.CompilerParams(
            dimension_semantics=("arbitrary", "arbitrary"),
        ),
        interpret=interpret,
    )(embeds, W1, b1, W2, b2_row)


def kernel(X, emb, W1, b1, W2, b2):
    idx = X.reshape(-1).astype(jnp.int32)
    rows = jnp.take(emb, idx, axis=0)
    embeds = rows.reshape(BATCH, CONTEXT * EMBED_DIM)
    return _fused_logsoftmax(embeds, W1, b1.reshape(1, HIDDEN),
                             b2.reshape(1, VOCAB), W2)


# speedup vs baseline: 1.0345x; 1.0345x over previous
import functools

import jax
import jax.numpy as jnp
from jax import lax
from jax.experimental import pallas as pl
from jax.experimental.pallas import tpu as pltpu

VOCAB = 100000
EMBED_DIM = 64
CONTEXT = 4
BATCH = 1024
HIDDEN = 128

VT = 2048
NT = (VOCAB + VT - 1) // VT

LOG2E = 1.4426950408889634
LN2 = 0.6931471805599453
NEG_BIG = -1e30


def _fused_body(embeds_ref, w1_ref, b1_ref, w2_ref, b2_ref, out_ref,
                h_ref, h2_ref, w2s_ref, m_ref, s_ref, mls_ref):
    p = pl.program_id(0)
    t = pl.program_id(1)

    @pl.when((p == 0) & (t == 0))
    def _init():
        e = embeds_ref[...].astype(jnp.bfloat16)
        w1 = w1_ref[...].astype(jnp.bfloat16)
        hf = lax.dot_general(e, w1, (((1,), (1,)), ((), ())),
                             preferred_element_type=jnp.float32)
        hf = jnp.maximum(hf + b1_ref[...], 0.0)
        h_ref[...] = hf.astype(jnp.bfloat16)
        h2_ref[...] = (hf * LOG2E).astype(jnp.bfloat16)
        m_ref[...] = jnp.full_like(m_ref, NEG_BIG)
        s_ref[...] = jnp.zeros_like(s_ref)

    @pl.when(p == 0)
    def _stats():
        row = t * VT + lax.broadcasted_iota(jnp.int32, (VT, 1), 0)
        w2bf = jnp.where(row < VOCAB, w2_ref[...], 0.0).astype(jnp.bfloat16)
        w2s_ref[pl.ds(t * VT, VT), :] = w2bf
        col = t * VT + lax.broadcasted_iota(jnp.int32, (1, VT), 1)
        b22 = jnp.where(col < VOCAB, b2_ref[...] * LOG2E, NEG_BIG)
        lg2 = lax.dot_general(h2_ref[...], w2bf, (((1,), (1,)), ((), ())),
                              preferred_element_type=jnp.float32) + b22
        tile_max = jnp.max(lg2, axis=1, keepdims=True)
        m_old = m_ref[...]
        m_new = jnp.maximum(m_old, tile_max)
        s_ref[...] = (s_ref[...] * jnp.exp2(m_old - m_new)
                      + jnp.sum(jnp.exp2(lg2 - m_new), axis=1, keepdims=True))
        m_ref[...] = m_new

    @pl.when(p == 1)
    def _write():
        @pl.when(t == 0)
        def _finalize():
            mls_ref[...] = m_ref[...] * LN2 + jnp.log(s_ref[...])

        w2bf = w2s_ref[pl.ds(t * VT, VT), :]
        logits = lax.dot_general(h_ref[...], w2bf, (((1,), (1,)), ((), ())),
                                 preferred_element_type=jnp.float32)
        out_ref[...] = (logits + b2_ref[...]) - mls_ref[...]


def _fused_logsoftmax(embeds, W1, b1, b2_row, W2, *, interpret=False):
    return pl.pallas_call(
        _fused_body,
        grid=(2, NT),
        in_specs=[
            pl.BlockSpec((BATCH, EMBED_DIM * CONTEXT), lambda p, t: (0, 0)),
            pl.BlockSpec((HIDDEN, EMBED_DIM * CONTEXT), lambda p, t: (0, 0)),
            pl.BlockSpec((1, HIDDEN), lambda p, t: (0, 0)),
            pl.BlockSpec((VT, HIDDEN), lambda p, t: (t * (1 - p), 0)),
            pl.BlockSpec((1, VT), lambda p, t: (0, t)),
        ],
        out_specs=pl.BlockSpec((BATCH, VT), lambda p, t: (0, p * t)),
        out_shape=jax.ShapeDtypeStruct((BATCH, VOCAB), jnp.float32),
        scratch_shapes=[
            pltpu.VMEM((BATCH, HIDDEN), jnp.bfloat16),
            pltpu.VMEM((BATCH, HIDDEN), jnp.bfloat16),
            pltpu.VMEM((NT * VT, HIDDEN), jnp.bfloat16),
            pltpu.VMEM((BATCH, 1), jnp.float32),
            pltpu.VMEM((BATCH, 1), jnp.float32),
            pltpu.VMEM((BATCH, 1), jnp.float32),
        ],
        compiler_params=pltpu.CompilerParams(
            dimension_semantics=("arbitrary", "arbitrary"),
        ),
        interpret=interpret,
    )(embeds, W1, b1, W2, b2_row)


def kernel(X, emb, W1, b1, W2, b2):
    idx = X.reshape(-1).astype(jnp.int32)
    rows = jnp.take(emb, idx, axis=0)
    embeds = rows.reshape(BATCH, CONTEXT * EMBED_DIM)
    return _fused_logsoftmax(embeds, W1, b1.reshape(1, HIDDEN),
                             b2.reshape(1, VOCAB), W2)
